# Initial kernel scaffold; baseline (speedup 1.0000x reference)
import jax
import jax.numpy as jnp
from jax.experimental import pallas as pl


def kernel(x, edge_index, W1, b1, W2, b2):
    raise NotImplementedError("write your pallas kernel here")


# speedup vs baseline: 19.7512x; 19.7512x over previous
import functools

import jax
import jax.numpy as jnp
from jax import lax
from jax.experimental import pallas as pl
from jax.experimental.pallas import tpu as pltpu
from jax.experimental.pallas import tpu_sc as plsc

_NC = 2
_NS = 16
_NW = _NC * _NS
_C = 80
_RB = 1280


def _mesh():
    return plsc.VectorSubcoreMesh(
        core_axis_name="c", subcore_axis_name="s",
        num_cores=_NC, num_subcores=_NS)


def _deg_kernel(N1, n_chunks):
    rpt = N1 // _NS

    @functools.partial(
        pl.kernel, mesh=_mesh(),
        out_type=jax.ShapeDtypeStruct((_NC, _NS, rpt), jnp.float32),
        scratch_types=[
            pltpu.VMEM((n_chunks, _C), jnp.int32),
            pltpu.VMEM((_C,), jnp.float32),
            pltpu.VMEM_SHARED((N1,), jnp.float32),
        ],
    )
    def k(dst_hbm, z1_hbm, degp, didx, ones_v, deg_sh):
        c = lax.axis_index("c")
        s = lax.axis_index("s")
        wid = c * _NS + s
        for i in range(_C // 16):
            ones_v[pl.ds(i * 16, 16)] = jnp.ones((16,), jnp.float32)
        pltpu.sync_copy(z1_hbm, deg_sh.at[pl.ds(s * rpt, rpt)])
        pltpu.sync_copy(dst_hbm.at[wid], didx)
        plsc.subcore_barrier()

        def body(j, carry):
            pltpu.sync_copy(ones_v, deg_sh.at[didx.at[j]], add=True)
            return carry
        lax.fori_loop(0, n_chunks, body, 0)
        plsc.subcore_barrier()
        pltpu.sync_copy(deg_sh.at[pl.ds(s * rpt, rpt)], degp.at[c, s])

    return k


def _agg_kernel(N1, D, n_chunks):
    rpt = N1 // _NS

    @functools.partial(
        pl.kernel, mesh=_mesh(),
        out_type=jax.ShapeDtypeStruct((_NC, _NS, rpt, D), jnp.float32),
        scratch_types=[
            pltpu.VMEM((n_chunks, _C), jnp.int32),
            pltpu.VMEM((n_chunks, _C), jnp.int32),
            pltpu.VMEM((_C, D), jnp.float32),
            pltpu.VMEM_SHARED((N1, D), jnp.float32),
            pltpu.SemaphoreType.DMA,
        ],
    )
    def k(hs_hbm, src_hbm, dst_hbm, z_hbm, accp, sidx, didx, rows_v, acc_sh, sem):
        c = lax.axis_index("c")
        s = lax.axis_index("s")
        wid = c * _NS + s
        pltpu.sync_copy(z_hbm, acc_sh.at[pl.ds(s * rpt, rpt)])
        pltpu.sync_copy(src_hbm.at[wid], sidx)
        pltpu.sync_copy(dst_hbm.at[wid], didx)
        plsc.subcore_barrier()

        def body(j, carry):
            pltpu.async_copy(hs_hbm.at[sidx.at[j]], rows_v, sem).wait()
            pltpu.sync_copy(rows_v, acc_sh.at[didx.at[j]], add=True)
            return carry
        lax.fori_loop(0, n_chunks, body, 0)
        plsc.subcore_barrier()
        pltpu.sync_copy(acc_sh.at[pl.ds(s * rpt, rpt)], accp.at[c, s])

    return k


def _tc1_body(x_ref, w_ref, deg_ref, hs_ref):
    dv = deg_ref[...]
    dinv = lax.rsqrt(dv[:, 0:1] + dv[:, 1:2] + 1.0)
    h = jnp.dot(x_ref[...], w_ref[...], preferred_element_type=jnp.float32)
    hs_ref[...] = h * dinv


def _tc2_body(acc_ref, hs_ref, deg_ref, b_ref, w_ref, hs2_ref):
    dv = deg_ref[...]
    dinv = lax.rsqrt(dv[:, 0:1] + dv[:, 1:2] + 1.0)
    av = acc_ref[...]
    a = av[0] + av[1] + hs_ref[...]
    z = jnp.maximum(dinv * a + b_ref[...], 0.0)
    h2 = jnp.dot(z, w_ref[...], preferred_element_type=jnp.float32)
    hs2_ref[...] = h2 * dinv


def _tc3_body(acc_ref, hs_ref, deg_ref, b_ref, out_ref):
    dv = deg_ref[...]
    dinv = lax.rsqrt(dv[:, 0:1] + dv[:, 1:2] + 1.0)
    av = acc_ref[...]
    a = av[0] + av[1] + hs_ref[...]
    out_ref[...] = dinv * a + b_ref[...]


def kernel(x, edge_index, W1, b1, W2, b2):
    N, D = x.shape
    H = W1.shape[1]
    E = edge_index.shape[1]

    n_chunks = -(-E // (_NW * _C))
    Ep = _NW * _C * n_chunks
    rpt = -(-(N + 1) // _NS)
    rpt = -(-rpt // 64) * 64
    N1 = _NS * rpt

    src = edge_index[0]
    dst = edge_index[1]
    if Ep != E:
        src = jnp.concatenate([src, jnp.zeros((Ep - E,), src.dtype)])
        dst = jnp.concatenate([dst, jnp.full((Ep - E,), N, dst.dtype)])
    src3 = src.reshape(_NW, n_chunks, _C)
    dst3 = dst.reshape(_NW, n_chunks, _C)

    z1 = jnp.zeros((rpt,), jnp.float32)
    zr = jnp.zeros((rpt, H), jnp.float32)

    degp = _deg_kernel(N1, n_chunks)(dst3, z1)
    degp_t = degp.reshape(_NC, N1).T

    grid = (pl.cdiv(N, _RB),)
    deg_spec = pl.BlockSpec((_RB, _NC), lambda i: (i, 0))
    row_spec = pl.BlockSpec((_RB, H), lambda i: (i, 0))
    acc_spec = pl.BlockSpec((_NC, _RB, H), lambda i: (0, i, 0))
    row_shape = jax.ShapeDtypeStruct((N, H), jnp.float32)
    b1r = b1.reshape(1, H)
    b2r = b2.reshape(1, H)

    hs1 = pl.pallas_call(
        _tc1_body, grid=grid,
        in_specs=[pl.BlockSpec((_RB, D), lambda i: (i, 0)),
                  pl.BlockSpec((D, H), lambda i: (0, 0)),
                  deg_spec],
        out_specs=row_spec,
        out_shape=row_shape,
    )(x, W1, degp_t)

    agg = _agg_kernel(N1, H, n_chunks)
    acc1 = agg(hs1, src3, dst3, zr).reshape(_NC, N1, H)

    hs2 = pl.pallas_call(
        _tc2_body, grid=grid,
        in_specs=[acc_spec, row_spec, deg_spec,
                  pl.BlockSpec((1, H), lambda i: (0, 0)),
                  pl.BlockSpec((H, H), lambda i: (0, 0))],
        out_specs=row_spec,
        out_shape=row_shape,
    )(acc1, hs1, degp_t, b1r, W2)

    acc2 = agg(hs2, src3, dst3, zr).reshape(_NC, N1, H)

    out = pl.pallas_call(
        _tc3_body, grid=grid,
        in_specs=[acc_spec, row_spec, deg_spec,
                  pl.BlockSpec((1, H), lambda i: (0, 0))],
        out_specs=row_spec,
        out_shape=row_shape,
    )(acc2, hs2, degp_t, b2r)

    return out
